# baseline (speedup 1.0000x reference)
import functools

import jax
import jax.numpy as jnp
from jax import lax
from jax.experimental import pallas as pl
from jax.experimental.pallas import tpu as pltpu
from jax.experimental.pallas import tpu_sc as plsc

_N = 10000
_E = 160000
_FI = 128
_H = 256
_HH = 128
_NG = 64
_NT = 16
_NC = 2
_EPT = _E // _NT
_CHP = 128
_NCHP = 80
_EPTP = _CHP * _NCHP
_NPAD = _EPTP - _EPT
_NTRASH = 16
_PACK = 32768
_RPT = 624
_TAIL0 = _RPT * _NT
_TAILN = _N - _TAIL0
_BN = 1000
_NB = _N // _BN
_F32 = jnp.float32


def _sc_mesh():
    return plsc.VectorSubcoreMesh(
        core_axis_name="c", subcore_axis_name="s", num_cores=_NC, num_subcores=_NT
    )


def _sc_degree_body(dst_hbm, ones_ch_hbm, init_hbm, out_hbm, acc, dstv, onesv, gsem):
    del gsem
    c = lax.axis_index("c")
    s = lax.axis_index("s")
    pltpu.sync_copy(dst_hbm.at[s], dstv)
    pltpu.sync_copy(ones_ch_hbm, onesv)
    pltpu.sync_copy(init_hbm.at[pl.ds(c * _RPT, _RPT)], acc.at[pl.ds(s * _RPT, _RPT)])

    @pl.when(s == _NT - 1)
    def _():
        pltpu.sync_copy(
            init_hbm.at[pl.ds(c * _RPT, _TAILN)], acc.at[pl.ds(_TAIL0, _TAILN)]
        )

    plsc.subcore_barrier()

    def step(j, carry):
        pltpu.sync_copy(onesv, acc.at[dstv.at[j]], add=True)
        return carry

    half = _NCHP // _NC
    lax.fori_loop(c * half, (c + 1) * half, step, 0)
    plsc.subcore_barrier()
    coff = c * _N
    pltpu.sync_copy(
        acc.at[pl.ds(s * _RPT, _RPT)], out_hbm.at[pl.ds(coff + s * _RPT, _RPT)]
    )

    @pl.when(s == _NT - 1)
    def _():
        pltpu.sync_copy(
            acc.at[pl.ds(_TAIL0, _TAILN)], out_hbm.at[pl.ds(coff + _TAIL0, _TAILN)]
        )


def _sc_degree(dstp3, ones_ch, init2):
    return pl.kernel(
        _sc_degree_body,
        out_type=jax.ShapeDtypeStruct((_NC * _N, _HH), _F32),
        mesh=_sc_mesh(),
        scratch_types=[
            pltpu.VMEM_SHARED((_N + _NTRASH, _HH), _F32),
            pltpu.VMEM((_NCHP, _CHP), jnp.int32),
            pltpu.VMEM((_CHP, _HH), _F32),
            pltpu.SemaphoreType.DMA,
        ],
    )(dstp3, ones_ch, init2)


def _sc_edge_body(
    hs_hbm, packed_hbm, out_hbm, acc, packv,
    srcb0, srcb1, dstb0, dstb1, rows0, rows1, sem0, sem1,
):
    c = lax.axis_index("c")
    s = lax.axis_index("s")
    coff = c * _N
    pltpu.sync_copy(packed_hbm.at[s], packv)
    pltpu.sync_copy(
        hs_hbm.at[pl.ds(coff + s * _RPT, _RPT)], acc.at[pl.ds(s * _RPT, _RPT)]
    )

    @pl.when(s == _NT - 1)
    def _():
        pltpu.sync_copy(
            hs_hbm.at[pl.ds(coff + _TAIL0, _TAILN)], acc.at[pl.ds(_TAIL0, _TAILN)]
        )

    rows = (rows0, rows1)
    srcb = (srcb0, srcb1)
    dstb = (dstb0, dstb1)
    sems = (sem0, sem1)

    def unpack(j, b):
        for k in range(_CHP // 16):
            v = packv[j, pl.ds(16 * k, 16)]
            srcb[b][pl.ds(16 * k, 16)] = (v & (_PACK - 1)) + coff
            dstb[b][pl.ds(16 * k, 16)] = lax.shift_right_logical(v, 15)

    plsc.subcore_barrier()
    for b in range(2):
        unpack(b, b)
        pltpu.async_copy(hs_hbm.at[srcb[b]], rows[b], sems[b])

    def outer(i, carry):
        j = i * 2
        for b in range(2):
            pltpu.make_async_copy(hs_hbm.at[srcb[b]], rows[b], sems[b]).wait()
            pltpu.sync_copy(rows[b], acc.at[dstb[b]], add=True)
            unpack(j + b + 2, b)
            pltpu.async_copy(hs_hbm.at[srcb[b]], rows[b], sems[b])
        return carry

    lax.fori_loop(0, (_NCHP - 2) // 2, outer, 0)
    for b in range(2):
        pltpu.make_async_copy(hs_hbm.at[srcb[b]], rows[b], sems[b]).wait()
        pltpu.sync_copy(rows[b], acc.at[dstb[b]], add=True)
    plsc.subcore_barrier()
    pltpu.sync_copy(
        acc.at[pl.ds(s * _RPT, _RPT)], out_hbm.at[pl.ds(coff + s * _RPT, _RPT)]
    )

    @pl.when(s == _NT - 1)
    def _():
        pltpu.sync_copy(
            acc.at[pl.ds(_TAIL0, _TAILN)],
            out_hbm.at[pl.ds(coff + _TAIL0, _TAILN)],
        )


def _sc_edge(hs_flat, packed3):
    return pl.kernel(
        _sc_edge_body,
        out_type=jax.ShapeDtypeStruct((_NC * _N, _HH), _F32),
        mesh=_sc_mesh(),
        scratch_types=[
            pltpu.VMEM_SHARED((_N + _NTRASH, _HH), _F32),
            pltpu.VMEM((_NCHP, _CHP), jnp.int32),
            pltpu.VMEM((_CHP,), jnp.int32),
            pltpu.VMEM((_CHP,), jnp.int32),
            pltpu.VMEM((_CHP,), jnp.int32),
            pltpu.VMEM((_CHP,), jnp.int32),
            pltpu.VMEM((_CHP, _HH), _F32),
            pltpu.VMEM((_CHP, _HH), _F32),
            pltpu.SemaphoreType.DMA,
            pltpu.SemaphoreType.DMA,
        ],
    )(hs_flat, packed3)


def _dot(a, b):
    return jnp.dot(
        a, b, preferred_element_type=_F32, precision=lax.Precision.HIGHEST
    )


def _tc_l0_body(x_ref, aw_ref, ab_ref, w_ref, b_ref, deg_ref, out_ref):
    h = _dot(x_ref[...], aw_ref[...]) + ab_ref[...]
    hh = _dot(h, w_ref[...]) + b_ref[...]
    rsq = lax.rsqrt(deg_ref[0] + deg_ref[1])
    out_ref[0] = hh[:, :_HH] * rsq
    out_ref[1] = hh[:, _HH:] * rsq


def _tc_l0(x, aw, ab, w, b, deg_rep):
    return pl.pallas_call(
        _tc_l0_body,
        grid=(_NB,),
        in_specs=[
            pl.BlockSpec((_BN, _FI), lambda i: (i, 0)),
            pl.BlockSpec((_FI, _H), lambda i: (0, 0)),
            pl.BlockSpec((1, _H), lambda i: (0, 0)),
            pl.BlockSpec((_H, _H), lambda i: (0, 0)),
            pl.BlockSpec((1, _H), lambda i: (0, 0)),
            pl.BlockSpec((_NC, _BN, _HH), lambda i: (0, i, 0)),
        ],
        out_specs=pl.BlockSpec((_NC, _BN, _HH), lambda i: (0, i, 0)),
        out_shape=jax.ShapeDtypeStruct((_NC, _N, _HH), _F32),
    )(x, aw, ab.reshape(1, _H), w, b.reshape(1, _H), deg_rep)


def _bn_coeffs(stats, g, be):
    mean = stats[0:1] * (1.0 / _N)
    var = stats[1:2] * (1.0 / _N) - mean * mean
    inv = lax.rsqrt(var + 1e-5)
    scale = inv * g
    shift = be - mean * scale
    return scale, shift


def _stats_update(stats_ref, i, y0, y1):
    s_row = jnp.concatenate(
        [jnp.sum(y0, axis=0, keepdims=True), jnp.sum(y1, axis=0, keepdims=True)],
        axis=1,
    )
    q_row = jnp.concatenate(
        [
            jnp.sum(y0 * y0, axis=0, keepdims=True),
            jnp.sum(y1 * y1, axis=0, keepdims=True),
        ],
        axis=1,
    )
    upd = jnp.concatenate([s_row, q_row], axis=0)

    @pl.when(i == 0)
    def _():
        stats_ref[...] = upd

    @pl.when(i > 0)
    def _():
        stats_ref[...] = stats_ref[...] + upd


def _tc_bn_layer_body(raw_ref, deg_ref, g_ref, be_ref, w_ref, b_ref, out_ref,
                      stats_ref):
    p = pl.program_id(0)
    i = pl.program_id(1)
    rsq = lax.rsqrt(deg_ref[0] + deg_ref[1])
    y0 = raw_ref[0] * rsq
    y1 = raw_ref[1] * rsq

    @pl.when(p == 0)
    def _():
        _stats_update(stats_ref, i, y0, y1)

    @pl.when(p == 1)
    def _():
        scale, shift = _bn_coeffs(stats_ref[...], g_ref[...], be_ref[...])
        h0 = jnp.maximum(y0 * scale[:, :_HH] + shift[:, :_HH], 0.0)
        h1 = jnp.maximum(y1 * scale[:, _HH:] + shift[:, _HH:], 0.0)
        hh = _dot(h0, w_ref[0]) + _dot(h1, w_ref[1]) + b_ref[...]
        out_ref[0] = hh[:, :_HH] * rsq
        out_ref[1] = hh[:, _HH:] * rsq


def _tc_bn_layer(raw2, deg_rep, g, be, w, b):
    return pl.pallas_call(
        _tc_bn_layer_body,
        grid=(2, _NB),
        in_specs=[
            pl.BlockSpec((_NC, _BN, _HH), lambda p, i: (0, i, 0)),
            pl.BlockSpec((_NC, _BN, _HH), lambda p, i: (0, i, 0)),
            pl.BlockSpec((1, _H), lambda p, i: (0, 0)),
            pl.BlockSpec((1, _H), lambda p, i: (0, 0)),
            pl.BlockSpec((_NC, _HH, _H), lambda p, i: (0, 0, 0)),
            pl.BlockSpec((1, _H), lambda p, i: (0, 0)),
        ],
        out_specs=pl.BlockSpec((_NC, _BN, _HH), lambda p, i: (0, i, 0)),
        out_shape=jax.ShapeDtypeStruct((_NC, _N, _HH), _F32),
        scratch_shapes=[pltpu.VMEM((2, _H), _F32)],
    )(
        raw2,
        deg_rep,
        g.reshape(1, _H),
        be.reshape(1, _H),
        w.reshape(_NC, _HH, _H),
        b.reshape(1, _H),
    )


def _tc_pool_pred_body(
    raw_ref, deg_ref, g_ref, be_ref, batch_ref, w1_ref, b1_ref,
    w2_ref, b2_ref, out_ref, stats_ref, pool_ref,
):
    p = pl.program_id(0)
    i = pl.program_id(1)
    rsq = lax.rsqrt(deg_ref[0] + deg_ref[1])
    y0 = raw_ref[0] * rsq
    y1 = raw_ref[1] * rsq

    @pl.when(p == 0)
    def _():
        _stats_update(stats_ref, i, y0, y1)

    @pl.when(p == 1)
    def _():
        scale, shift = _bn_coeffs(stats_ref[...], g_ref[...], be_ref[...])
        h0 = jnp.maximum(y0 * scale[:, :_HH] + shift[:, :_HH], 0.0)
        h1 = jnp.maximum(y1 * scale[:, _HH:] + shift[:, _HH:], 0.0)
        seg = batch_ref[0]
        oh = (lax.broadcasted_iota(jnp.int32, (_NG, _BN), 0) == seg).astype(_F32)
        upd = jnp.concatenate([_dot(oh, h0), _dot(oh, h1)], axis=1)

        @pl.when(i == 0)
        def _():
            pool_ref[...] = upd

        @pl.when(i > 0)
        def _():
            pool_ref[...] = pool_ref[...] + upd

        @pl.when(i == _NB - 1)
        def _():
            hp = jnp.maximum(_dot(pool_ref[...], w1_ref[...]) + b1_ref[...], 0.0)
            out_ref[...] = _dot(hp, w2_ref[...]) + b2_ref[...]


def _tc_pool_pred(raw2, deg_rep, g, be, batch3, w1, b1, w2p, b2p):
    return pl.pallas_call(
        _tc_pool_pred_body,
        grid=(2, _NB),
        in_specs=[
            pl.BlockSpec((_NC, _BN, _HH), lambda p, i: (0, i, 0)),
            pl.BlockSpec((_NC, _BN, _HH), lambda p, i: (0, i, 0)),
            pl.BlockSpec((1, _H), lambda p, i: (0, 0)),
            pl.BlockSpec((1, _H), lambda p, i: (0, 0)),
            pl.BlockSpec((1, 1, _BN), lambda p, i: (i, 0, 0)),
            pl.BlockSpec((_H, _H), lambda p, i: (0, 0)),
            pl.BlockSpec((1, _H), lambda p, i: (0, 0)),
            pl.BlockSpec((_H, _HH), lambda p, i: (0, 0)),
            pl.BlockSpec((1, _HH), lambda p, i: (0, 0)),
        ],
        out_specs=pl.BlockSpec((_NG, _HH), lambda p, i: (0, 0)),
        out_shape=jax.ShapeDtypeStruct((_NG, _HH), _F32),
        scratch_shapes=[pltpu.VMEM((2, _H), _F32), pltpu.VMEM((_NG, _H), _F32)],
    )(
        raw2,
        deg_rep,
        g.reshape(1, _H),
        be.reshape(1, _H),
        batch3,
        w1,
        b1.reshape(1, _H),
        w2p,
        b2p.reshape(1, _HH),
    )


def kernel(x, edge_index, batch, atom_W, atom_b,
           conv_W0, conv_b0, bn_g0, bn_b0,
           conv_W1, conv_b1, bn_g1, bn_b1,
           conv_W2, conv_b2, bn_g2, bn_b2,
           pred_W1, pred_b1, pred_W2, pred_b2):
    src = edge_index[0].astype(jnp.int32).reshape(_NT, _EPT)
    dst = edge_index[1].astype(jnp.int32).reshape(_NT, _EPT)
    pad_src = (jnp.arange(_NPAD, dtype=jnp.int32)[None, :] * 61
               + jnp.arange(_NT, dtype=jnp.int32)[:, None] * 607) % _N
    pad_dst = _N + (jnp.arange(_NPAD, dtype=jnp.int32)[None, :] % _NTRASH
                    ) + jnp.zeros((_NT, 1), jnp.int32)
    srcp = jnp.concatenate([src, pad_src], axis=1)
    dstp = jnp.concatenate([dst, pad_dst], axis=1)
    packed3 = (dstp * _PACK + srcp).reshape(_NT, _NCHP, _CHP)
    dstp3 = dstp.reshape(_NT, _NCHP, _CHP)
    ones_ch = jnp.ones((_CHP, _HH), _F32)
    init2 = jnp.concatenate(
        [jnp.ones((_RPT, _HH), _F32), jnp.zeros((_RPT, _HH), _F32)], axis=0
    )
    batch3 = batch.astype(jnp.int32).reshape(_NB, 1, _BN)
    w2p = jnp.pad(pred_W2, ((0, 0), (0, _HH - pred_W2.shape[1])))
    b2p = jnp.pad(pred_b2, (0, _HH - pred_b2.shape[0]))

    deg_rep = _sc_degree(dstp3, ones_ch, init2).reshape(_NC, _N, _HH)

    hs = _tc_l0(x, atom_W, atom_b, conv_W0, conv_b0, deg_rep)
    raw = _sc_edge(hs.reshape(_NC * _N, _HH), packed3).reshape(_NC, _N, _HH)
    hs = _tc_bn_layer(raw, deg_rep, bn_g0, bn_b0, conv_W1, conv_b1)
    raw = _sc_edge(hs.reshape(_NC * _N, _HH), packed3).reshape(_NC, _N, _HH)
    hs = _tc_bn_layer(raw, deg_rep, bn_g1, bn_b1, conv_W2, conv_b2)
    raw = _sc_edge(hs.reshape(_NC * _N, _HH), packed3).reshape(_NC, _N, _HH)
    y = _tc_pool_pred(raw, deg_rep, bn_g2, bn_b2, batch3,
                      pred_W1, pred_b1, w2p, b2p)
    return y[:, 0]


# speedup vs baseline: 1.0076x; 1.0076x over previous
import functools

import jax
import jax.numpy as jnp
from jax import lax
from jax.experimental import pallas as pl
from jax.experimental.pallas import tpu as pltpu
from jax.experimental.pallas import tpu_sc as plsc

_N = 10000
_E = 160000
_FI = 128
_H = 256
_HH = 128
_NG = 64
_NT = 16
_NC = 2
_EPT = _E // _NT
_CHP = 128
_NCHP = 80
_EPTP = _CHP * _NCHP
_NPAD = _EPTP - _EPT
_NTRASH = 16
_PACK = 32768
_RPT = 624
_TAIL0 = _RPT * _NT
_TAILN = _N - _TAIL0
_BN = 1000
_NB = _N // _BN
_F32 = jnp.float32


def _sc_mesh():
    return plsc.VectorSubcoreMesh(
        core_axis_name="c", subcore_axis_name="s", num_cores=_NC, num_subcores=_NT
    )


def _sc_degree_body(dst_hbm, ones_ch_hbm, init_hbm, out_hbm, acc, dstv, onesv, gsem):
    del gsem
    c = lax.axis_index("c")
    s = lax.axis_index("s")
    pltpu.sync_copy(dst_hbm.at[s], dstv)
    pltpu.sync_copy(ones_ch_hbm, onesv)
    pltpu.sync_copy(init_hbm.at[pl.ds(c * _RPT, _RPT)], acc.at[pl.ds(s * _RPT, _RPT)])

    @pl.when(s == _NT - 1)
    def _():
        pltpu.sync_copy(
            init_hbm.at[pl.ds(c * _RPT, _TAILN)], acc.at[pl.ds(_TAIL0, _TAILN)]
        )

    plsc.subcore_barrier()

    def step(j, carry):
        pltpu.sync_copy(onesv, acc.at[dstv.at[j]], add=True)
        return carry

    half = _NCHP // _NC
    lax.fori_loop(c * half, (c + 1) * half, step, 0)
    plsc.subcore_barrier()
    coff = c * _N
    pltpu.sync_copy(
        acc.at[pl.ds(s * _RPT, _RPT)], out_hbm.at[pl.ds(coff + s * _RPT, _RPT)]
    )

    @pl.when(s == _NT - 1)
    def _():
        pltpu.sync_copy(
            acc.at[pl.ds(_TAIL0, _TAILN)], out_hbm.at[pl.ds(coff + _TAIL0, _TAILN)]
        )


def _sc_degree(dstp3, ones_ch, init2):
    return pl.kernel(
        _sc_degree_body,
        out_type=jax.ShapeDtypeStruct((_NC * _N, _HH), _F32),
        mesh=_sc_mesh(),
        scratch_types=[
            pltpu.VMEM_SHARED((_N + _NTRASH, _HH), _F32),
            pltpu.VMEM((_NCHP, _CHP), jnp.int32),
            pltpu.VMEM((_CHP, _HH), _F32),
            pltpu.SemaphoreType.DMA,
        ],
    )(dstp3, ones_ch, init2)


def _sc_edge_body(
    hs_hbm, packed_hbm, out_hbm, acc, packv,
    srcb0, srcb1, dstb0, dstb1, rows0, rows1, sem0, sem1,
):
    c = lax.axis_index("c")
    s = lax.axis_index("s")
    coff = c * _N
    pltpu.sync_copy(packed_hbm.at[s], packv)
    pltpu.sync_copy(
        hs_hbm.at[pl.ds(coff + s * _RPT, _RPT)], acc.at[pl.ds(s * _RPT, _RPT)]
    )

    @pl.when(s == _NT - 1)
    def _():
        pltpu.sync_copy(
            hs_hbm.at[pl.ds(coff + _TAIL0, _TAILN)], acc.at[pl.ds(_TAIL0, _TAILN)]
        )

    rows = (rows0, rows1)
    srcb = (srcb0, srcb1)
    dstb = (dstb0, dstb1)
    sems = (sem0, sem1)

    def unpack(j, b):
        for k in range(_CHP // 16):
            v = packv[j, pl.ds(16 * k, 16)]
            srcb[b][pl.ds(16 * k, 16)] = (v & (_PACK - 1)) + coff
            dstb[b][pl.ds(16 * k, 16)] = lax.shift_right_logical(v, 15)

    plsc.subcore_barrier()
    for b in range(2):
        unpack(b, b)
        pltpu.async_copy(hs_hbm.at[srcb[b]], rows[b], sems[b])

    def outer(i, carry):
        j = i * 2
        for b in range(2):
            pltpu.make_async_copy(hs_hbm.at[srcb[b]], rows[b], sems[b]).wait()
            pltpu.sync_copy(rows[b], acc.at[dstb[b]], add=True)
            unpack(j + b + 2, b)
            pltpu.async_copy(hs_hbm.at[srcb[b]], rows[b], sems[b])
        return carry

    lax.fori_loop(0, (_NCHP - 2) // 2, outer, 0)
    for b in range(2):
        pltpu.make_async_copy(hs_hbm.at[srcb[b]], rows[b], sems[b]).wait()
        pltpu.sync_copy(rows[b], acc.at[dstb[b]], add=True)
    plsc.subcore_barrier()
    pltpu.sync_copy(
        acc.at[pl.ds(s * _RPT, _RPT)], out_hbm.at[pl.ds(coff + s * _RPT, _RPT)]
    )

    @pl.when(s == _NT - 1)
    def _():
        pltpu.sync_copy(
            acc.at[pl.ds(_TAIL0, _TAILN)],
            out_hbm.at[pl.ds(coff + _TAIL0, _TAILN)],
        )


def _sc_edge(hs_flat, packed3):
    return pl.kernel(
        _sc_edge_body,
        out_type=jax.ShapeDtypeStruct((_NC * _N, _HH), _F32),
        mesh=_sc_mesh(),
        scratch_types=[
            pltpu.VMEM_SHARED((_N + _NTRASH, _HH), _F32),
            pltpu.VMEM((_NCHP, _CHP), jnp.int32),
            pltpu.VMEM((_CHP,), jnp.int32),
            pltpu.VMEM((_CHP,), jnp.int32),
            pltpu.VMEM((_CHP,), jnp.int32),
            pltpu.VMEM((_CHP,), jnp.int32),
            pltpu.VMEM((_CHP, _HH), _F32),
            pltpu.VMEM((_CHP, _HH), _F32),
            pltpu.SemaphoreType.DMA,
            pltpu.SemaphoreType.DMA,
        ],
    )(hs_flat, packed3)


def _dot(a, b):
    return jnp.dot(
        a, b, preferred_element_type=_F32, precision=lax.Precision.HIGHEST
    )


def _tc_l0_body(x_ref, aw_ref, ab_ref, w_ref, b_ref, deg_ref, out_ref):
    h = _dot(x_ref[...], aw_ref[...]) + ab_ref[...]
    hh = _dot(h, w_ref[...]) + b_ref[...]
    rsq = lax.rsqrt(deg_ref[0] + deg_ref[1])
    out_ref[0] = hh[:, :_HH] * rsq
    out_ref[1] = hh[:, _HH:] * rsq


def _tc_l0(x, aw, ab, w, b, deg_rep):
    return pl.pallas_call(
        _tc_l0_body,
        grid=(_NB,),
        in_specs=[
            pl.BlockSpec((_BN, _FI), lambda i: (i, 0)),
            pl.BlockSpec((_FI, _H), lambda i: (0, 0)),
            pl.BlockSpec((1, _H), lambda i: (0, 0)),
            pl.BlockSpec((_H, _H), lambda i: (0, 0)),
            pl.BlockSpec((1, _H), lambda i: (0, 0)),
            pl.BlockSpec((_NC, _BN, _HH), lambda i: (0, i, 0)),
        ],
        out_specs=pl.BlockSpec((_NC, _BN, _HH), lambda i: (0, i, 0)),
        out_shape=jax.ShapeDtypeStruct((_NC, _N, _HH), _F32),
    )(x, aw, ab.reshape(1, _H), w, b.reshape(1, _H), deg_rep)


def _bn_coeffs(stats, g, be):
    mean = stats[0:1] * (1.0 / _N)
    var = stats[1:2] * (1.0 / _N) - mean * mean
    inv = lax.rsqrt(var + 1e-5)
    scale = inv * g
    shift = be - mean * scale
    return scale, shift


def _stats_update(stats_ref, i, y0, y1):
    s_row = jnp.concatenate(
        [jnp.sum(y0, axis=0, keepdims=True), jnp.sum(y1, axis=0, keepdims=True)],
        axis=1,
    )
    q_row = jnp.concatenate(
        [
            jnp.sum(y0 * y0, axis=0, keepdims=True),
            jnp.sum(y1 * y1, axis=0, keepdims=True),
        ],
        axis=1,
    )
    upd = jnp.concatenate([s_row, q_row], axis=0)

    @pl.when(i == 0)
    def _():
        stats_ref[...] = upd

    @pl.when(i > 0)
    def _():
        stats_ref[...] = stats_ref[...] + upd


def _tc_bn_layer_body(raw_ref, deg_ref, g_ref, be_ref, w_ref, b_ref, out_ref,
                      stats_ref):
    p = pl.program_id(0)
    i = pl.program_id(1)
    rsq = lax.rsqrt(deg_ref[0] + deg_ref[1])
    y0 = raw_ref[0] * rsq
    y1 = raw_ref[1] * rsq

    @pl.when(p == 0)
    def _():
        _stats_update(stats_ref, i, y0, y1)

    @pl.when(p == 1)
    def _():
        scale, shift = _bn_coeffs(stats_ref[...], g_ref[...], be_ref[...])
        h0 = jnp.maximum(y0 * scale[:, :_HH] + shift[:, :_HH], 0.0)
        h1 = jnp.maximum(y1 * scale[:, _HH:] + shift[:, _HH:], 0.0)
        hh = _dot(h0, w_ref[0]) + _dot(h1, w_ref[1]) + b_ref[...]
        out_ref[0] = hh[:, :_HH] * rsq
        out_ref[1] = hh[:, _HH:] * rsq


def _tc_bn_layer(raw2, deg_rep, g, be, w, b):
    return pl.pallas_call(
        _tc_bn_layer_body,
        grid=(2, _NB),
        in_specs=[
            pl.BlockSpec((_NC, _BN, _HH), lambda p, i: (0, i, 0)),
            pl.BlockSpec((_NC, _BN, _HH), lambda p, i: (0, i, 0)),
            pl.BlockSpec((1, _H), lambda p, i: (0, 0)),
            pl.BlockSpec((1, _H), lambda p, i: (0, 0)),
            pl.BlockSpec((_NC, _HH, _H), lambda p, i: (0, 0, 0)),
            pl.BlockSpec((1, _H), lambda p, i: (0, 0)),
        ],
        out_specs=pl.BlockSpec((_NC, _BN, _HH), lambda p, i: (0, i * p, 0)),
        out_shape=jax.ShapeDtypeStruct((_NC, _N, _HH), _F32),
        scratch_shapes=[pltpu.VMEM((2, _H), _F32)],
    )(
        raw2,
        deg_rep,
        g.reshape(1, _H),
        be.reshape(1, _H),
        w.reshape(_NC, _HH, _H),
        b.reshape(1, _H),
    )


def _tc_pool_pred_body(
    raw_ref, deg_ref, g_ref, be_ref, batch_ref, w1_ref, b1_ref,
    w2_ref, b2_ref, out_ref, stats_ref, pool_ref,
):
    p = pl.program_id(0)
    i = pl.program_id(1)
    rsq = lax.rsqrt(deg_ref[0] + deg_ref[1])
    y0 = raw_ref[0] * rsq
    y1 = raw_ref[1] * rsq

    @pl.when(p == 0)
    def _():
        _stats_update(stats_ref, i, y0, y1)

    @pl.when(p == 1)
    def _():
        scale, shift = _bn_coeffs(stats_ref[...], g_ref[...], be_ref[...])
        h0 = jnp.maximum(y0 * scale[:, :_HH] + shift[:, :_HH], 0.0)
        h1 = jnp.maximum(y1 * scale[:, _HH:] + shift[:, _HH:], 0.0)
        seg = batch_ref[0]
        oh = (lax.broadcasted_iota(jnp.int32, (_NG, _BN), 0) == seg).astype(_F32)
        upd = jnp.concatenate([_dot(oh, h0), _dot(oh, h1)], axis=1)

        @pl.when(i == 0)
        def _():
            pool_ref[...] = upd

        @pl.when(i > 0)
        def _():
            pool_ref[...] = pool_ref[...] + upd

        @pl.when(i == _NB - 1)
        def _():
            hp = jnp.maximum(_dot(pool_ref[...], w1_ref[...]) + b1_ref[...], 0.0)
            out_ref[...] = _dot(hp, w2_ref[...]) + b2_ref[...]


def _tc_pool_pred(raw2, deg_rep, g, be, batch3, w1, b1, w2p, b2p):
    return pl.pallas_call(
        _tc_pool_pred_body,
        grid=(2, _NB),
        in_specs=[
            pl.BlockSpec((_NC, _BN, _HH), lambda p, i: (0, i, 0)),
            pl.BlockSpec((_NC, _BN, _HH), lambda p, i: (0, i, 0)),
            pl.BlockSpec((1, _H), lambda p, i: (0, 0)),
            pl.BlockSpec((1, _H), lambda p, i: (0, 0)),
            pl.BlockSpec((1, 1, _BN), lambda p, i: (i, 0, 0)),
            pl.BlockSpec((_H, _H), lambda p, i: (0, 0)),
            pl.BlockSpec((1, _H), lambda p, i: (0, 0)),
            pl.BlockSpec((_H, _HH), lambda p, i: (0, 0)),
            pl.BlockSpec((1, _HH), lambda p, i: (0, 0)),
        ],
        out_specs=pl.BlockSpec((_NG, _HH), lambda p, i: (0, 0)),
        out_shape=jax.ShapeDtypeStruct((_NG, _HH), _F32),
        scratch_shapes=[pltpu.VMEM((2, _H), _F32), pltpu.VMEM((_NG, _H), _F32)],
    )(
        raw2,
        deg_rep,
        g.reshape(1, _H),
        be.reshape(1, _H),
        batch3,
        w1,
        b1.reshape(1, _H),
        w2p,
        b2p.reshape(1, _HH),
    )


def kernel(x, edge_index, batch, atom_W, atom_b,
           conv_W0, conv_b0, bn_g0, bn_b0,
           conv_W1, conv_b1, bn_g1, bn_b1,
           conv_W2, conv_b2, bn_g2, bn_b2,
           pred_W1, pred_b1, pred_W2, pred_b2):
    src = edge_index[0].astype(jnp.int32).reshape(_NT, _EPT)
    dst = edge_index[1].astype(jnp.int32).reshape(_NT, _EPT)
    pad_src = (jnp.arange(_NPAD, dtype=jnp.int32)[None, :] * 61
               + jnp.arange(_NT, dtype=jnp.int32)[:, None] * 607) % _N
    pad_dst = _N + (jnp.arange(_NPAD, dtype=jnp.int32)[None, :] % _NTRASH
                    ) + jnp.zeros((_NT, 1), jnp.int32)
    srcp = jnp.concatenate([src, pad_src], axis=1)
    dstp = jnp.concatenate([dst, pad_dst], axis=1)
    packed3 = (dstp * _PACK + srcp).reshape(_NT, _NCHP, _CHP)
    dstp3 = dstp.reshape(_NT, _NCHP, _CHP)
    ones_ch = jnp.ones((_CHP, _HH), _F32)
    init2 = jnp.concatenate(
        [jnp.ones((_RPT, _HH), _F32), jnp.zeros((_RPT, _HH), _F32)], axis=0
    )
    batch3 = batch.astype(jnp.int32).reshape(_NB, 1, _BN)
    w2p = jnp.pad(pred_W2, ((0, 0), (0, _HH - pred_W2.shape[1])))
    b2p = jnp.pad(pred_b2, (0, _HH - pred_b2.shape[0]))

    deg_rep = _sc_degree(dstp3, ones_ch, init2).reshape(_NC, _N, _HH)

    hs = _tc_l0(x, atom_W, atom_b, conv_W0, conv_b0, deg_rep)
    raw = _sc_edge(hs.reshape(_NC * _N, _HH), packed3).reshape(_NC, _N, _HH)
    hs = _tc_bn_layer(raw, deg_rep, bn_g0, bn_b0, conv_W1, conv_b1)
    raw = _sc_edge(hs.reshape(_NC * _N, _HH), packed3).reshape(_NC, _N, _HH)
    hs = _tc_bn_layer(raw, deg_rep, bn_g1, bn_b1, conv_W2, conv_b2)
    raw = _sc_edge(hs.reshape(_NC * _N, _HH), packed3).reshape(_NC, _N, _HH)
    y = _tc_pool_pred(raw, deg_rep, bn_g2, bn_b2, batch3,
                      pred_W1, pred_b1, w2p, b2p)
    return y[:, 0]


# speedup vs baseline: 1.0608x; 1.0528x over previous
import functools

import jax
import jax.numpy as jnp
from jax import lax
from jax.experimental import pallas as pl
from jax.experimental.pallas import tpu as pltpu
from jax.experimental.pallas import tpu_sc as plsc

_N = 10000
_E = 160000
_FI = 128
_H = 256
_HH = 128
_NG = 64
_NT = 16
_NC = 2
_EPT = _E // _NT
_CHP = 128
_NCHP = 80
_EPTP = _CHP * _NCHP
_NPAD = _EPTP - _EPT
_NTRASH = 16
_PACK = 32768
_RPT = 624
_TAIL0 = _RPT * _NT
_TAILN = _N - _TAIL0
_BN = 1000
_NB = _N // _BN
_F32 = jnp.float32


def _sc_mesh():
    return plsc.VectorSubcoreMesh(
        core_axis_name="c", subcore_axis_name="s", num_cores=_NC, num_subcores=_NT
    )


def _sc_degree_body(dst_hbm, ones_ch_hbm, init_hbm, out_hbm, acc, dstv, onesv, gsem):
    del gsem
    c = lax.axis_index("c")
    s = lax.axis_index("s")
    pltpu.sync_copy(dst_hbm.at[s], dstv)
    pltpu.sync_copy(ones_ch_hbm, onesv)
    pltpu.sync_copy(init_hbm.at[pl.ds(c * _RPT, _RPT)], acc.at[pl.ds(s * _RPT, _RPT)])

    @pl.when(s == _NT - 1)
    def _():
        pltpu.sync_copy(
            init_hbm.at[pl.ds(c * _RPT, _TAILN)], acc.at[pl.ds(_TAIL0, _TAILN)]
        )

    plsc.subcore_barrier()

    def step(j, carry):
        pltpu.sync_copy(onesv, acc.at[dstv.at[j]], add=True)
        return carry

    half = _NCHP // _NC
    lax.fori_loop(c * half, (c + 1) * half, step, 0)
    plsc.subcore_barrier()
    coff = c * _N
    pltpu.sync_copy(
        acc.at[pl.ds(s * _RPT, _RPT)], out_hbm.at[pl.ds(coff + s * _RPT, _RPT)]
    )

    @pl.when(s == _NT - 1)
    def _():
        pltpu.sync_copy(
            acc.at[pl.ds(_TAIL0, _TAILN)], out_hbm.at[pl.ds(coff + _TAIL0, _TAILN)]
        )


def _sc_degree(dstp3, ones_ch, init2):
    return pl.kernel(
        _sc_degree_body,
        out_type=jax.ShapeDtypeStruct((_NC * _N, _HH), _F32),
        mesh=_sc_mesh(),
        scratch_types=[
            pltpu.VMEM_SHARED((_N + _NTRASH, _HH), _F32),
            pltpu.VMEM((_NCHP, _CHP), jnp.int32),
            pltpu.VMEM((_CHP, _HH), _F32),
            pltpu.SemaphoreType.DMA,
        ],
    )(dstp3, ones_ch, init2)


def _sc_edge_body(
    hs_hbm, packed_hbm, out_hbm, acc, packv,
    srcb0, srcb1, dstb0, dstb1, rows0, rows1, sem0, sem1,
):
    c = lax.axis_index("c")
    s = lax.axis_index("s")
    coff = c * _N
    pltpu.sync_copy(packed_hbm.at[s], packv)
    pltpu.sync_copy(
        hs_hbm.at[pl.ds(coff + s * _RPT, _RPT)], acc.at[pl.ds(s * _RPT, _RPT)]
    )

    @pl.when(s == _NT - 1)
    def _():
        pltpu.sync_copy(
            hs_hbm.at[pl.ds(coff + _TAIL0, _TAILN)], acc.at[pl.ds(_TAIL0, _TAILN)]
        )

    rows = (rows0, rows1)
    srcb = (srcb0, srcb1)
    dstb = (dstb0, dstb1)
    sems = (sem0, sem1)

    def unpack(j, b):
        for k in range(_CHP // 16):
            v = packv[j, pl.ds(16 * k, 16)]
            srcb[b][pl.ds(16 * k, 16)] = (v & (_PACK - 1)) + coff
            dstb[b][pl.ds(16 * k, 16)] = lax.shift_right_logical(v, 15)

    plsc.subcore_barrier()
    for b in range(2):
        unpack(b, b)
        pltpu.async_copy(hs_hbm.at[srcb[b]], rows[b], sems[b])

    def outer(i, carry):
        j = i * 2
        for b in range(2):
            pltpu.make_async_copy(hs_hbm.at[srcb[b]], rows[b], sems[b]).wait()
            pltpu.sync_copy(rows[b], acc.at[dstb[b]], add=True)
            unpack(j + b + 2, b)
            pltpu.async_copy(hs_hbm.at[srcb[b]], rows[b], sems[b])
        return carry

    lax.fori_loop(0, (_NCHP - 2) // 2, outer, 0)
    for b in range(2):
        pltpu.make_async_copy(hs_hbm.at[srcb[b]], rows[b], sems[b]).wait()
        pltpu.sync_copy(rows[b], acc.at[dstb[b]], add=True)
    plsc.subcore_barrier()
    pltpu.sync_copy(
        acc.at[pl.ds(s * _RPT, _RPT)], out_hbm.at[pl.ds(coff + s * _RPT, _RPT)]
    )

    @pl.when(s == _NT - 1)
    def _():
        pltpu.sync_copy(
            acc.at[pl.ds(_TAIL0, _TAILN)],
            out_hbm.at[pl.ds(coff + _TAIL0, _TAILN)],
        )


def _sc_edge(hs_flat, packed3):
    return pl.kernel(
        _sc_edge_body,
        out_type=jax.ShapeDtypeStruct((_NC * _N, _HH), _F32),
        mesh=_sc_mesh(),
        scratch_types=[
            pltpu.VMEM_SHARED((_N + _NTRASH, _HH), _F32),
            pltpu.VMEM((_NCHP, _CHP), jnp.int32),
            pltpu.VMEM((_CHP,), jnp.int32),
            pltpu.VMEM((_CHP,), jnp.int32),
            pltpu.VMEM((_CHP,), jnp.int32),
            pltpu.VMEM((_CHP,), jnp.int32),
            pltpu.VMEM((_CHP, _HH), _F32),
            pltpu.VMEM((_CHP, _HH), _F32),
            pltpu.SemaphoreType.DMA,
            pltpu.SemaphoreType.DMA,
        ],
    )(hs_flat, packed3)


def _dot(a, b):
    ah = a.astype(jnp.bfloat16)
    al = (a - ah.astype(_F32)).astype(jnp.bfloat16)
    bh = b.astype(jnp.bfloat16)
    bl = (b - bh.astype(_F32)).astype(jnp.bfloat16)
    d = lambda u, v: jnp.dot(u, v, preferred_element_type=_F32)
    return d(ah, bl) + d(al, bh) + d(ah, bh)


def _tc_l0_body(x_ref, aw_ref, ab_ref, w_ref, b_ref, deg_ref, out_ref):
    h = _dot(x_ref[...], aw_ref[...]) + ab_ref[...]
    hh = _dot(h, w_ref[...]) + b_ref[...]
    rsq = 1.0 / jnp.sqrt(deg_ref[0] + deg_ref[1])
    out_ref[0] = hh[:, :_HH] * rsq
    out_ref[1] = hh[:, _HH:] * rsq


def _tc_l0(x, aw, ab, w, b, deg_rep):
    return pl.pallas_call(
        _tc_l0_body,
        grid=(_NB,),
        in_specs=[
            pl.BlockSpec((_BN, _FI), lambda i: (i, 0)),
            pl.BlockSpec((_FI, _H), lambda i: (0, 0)),
            pl.BlockSpec((1, _H), lambda i: (0, 0)),
            pl.BlockSpec((_H, _H), lambda i: (0, 0)),
            pl.BlockSpec((1, _H), lambda i: (0, 0)),
            pl.BlockSpec((_NC, _BN, _HH), lambda i: (0, i, 0)),
        ],
        out_specs=pl.BlockSpec((_NC, _BN, _HH), lambda i: (0, i, 0)),
        out_shape=jax.ShapeDtypeStruct((_NC, _N, _HH), _F32),
    )(x, aw, ab.reshape(1, _H), w, b.reshape(1, _H), deg_rep)


def _bn_coeffs(stats, g, be):
    m0 = stats[0:1] * (1.0 / _N)
    mean = stats[2:3] + m0
    var = stats[1:2] * (1.0 / _N) - m0 * m0
    inv = 1.0 / jnp.sqrt(var + 1e-5)
    scale = inv * g
    shift = be - mean * scale
    return scale, shift


def _stats_update(stats_ref, i, y0, y1):
    @pl.when(i == 0)
    def _():
        c_row = jnp.concatenate(
            [
                jnp.sum(y0, axis=0, keepdims=True),
                jnp.sum(y1, axis=0, keepdims=True),
            ],
            axis=1,
        ) * (1.0 / _BN)
        stats_ref[2:3] = c_row
        stats_ref[0:2] = jnp.zeros((2, _H), _F32)

    c = stats_ref[2:3]
    d0 = y0 - c[:, :_HH]
    d1 = y1 - c[:, _HH:]
    s_row = jnp.concatenate(
        [jnp.sum(d0, axis=0, keepdims=True), jnp.sum(d1, axis=0, keepdims=True)],
        axis=1,
    )
    q_row = jnp.concatenate(
        [
            jnp.sum(d0 * d0, axis=0, keepdims=True),
            jnp.sum(d1 * d1, axis=0, keepdims=True),
        ],
        axis=1,
    )
    stats_ref[0:2] = stats_ref[0:2] + jnp.concatenate([s_row, q_row], axis=0)


def _tc_bn_layer_body(raw_ref, deg_ref, g_ref, be_ref, w_ref, b_ref, out_ref,
                      stats_ref):
    p = pl.program_id(0)
    i = pl.program_id(1)
    rsq = 1.0 / jnp.sqrt(deg_ref[0] + deg_ref[1])
    y0 = raw_ref[0] * rsq
    y1 = raw_ref[1] * rsq

    @pl.when(p == 0)
    def _():
        _stats_update(stats_ref, i, y0, y1)

    @pl.when(p == 1)
    def _():
        scale, shift = _bn_coeffs(stats_ref[...], g_ref[...], be_ref[...])
        h0 = jnp.maximum(y0 * scale[:, :_HH] + shift[:, :_HH], 0.0)
        h1 = jnp.maximum(y1 * scale[:, _HH:] + shift[:, _HH:], 0.0)
        hh = _dot(jnp.concatenate([h0, h1], axis=1), w_ref[...]) + b_ref[...]
        out_ref[0] = hh[:, :_HH] * rsq
        out_ref[1] = hh[:, _HH:] * rsq


def _tc_bn_layer(raw2, deg_rep, g, be, w, b):
    return pl.pallas_call(
        _tc_bn_layer_body,
        grid=(2, _NB),
        in_specs=[
            pl.BlockSpec((_NC, _BN, _HH), lambda p, i: (0, i, 0)),
            pl.BlockSpec((_NC, _BN, _HH), lambda p, i: (0, i, 0)),
            pl.BlockSpec((1, _H), lambda p, i: (0, 0)),
            pl.BlockSpec((1, _H), lambda p, i: (0, 0)),
            pl.BlockSpec((_H, _H), lambda p, i: (0, 0)),
            pl.BlockSpec((1, _H), lambda p, i: (0, 0)),
        ],
        out_specs=pl.BlockSpec((_NC, _BN, _HH), lambda p, i: (0, i * p, 0)),
        out_shape=jax.ShapeDtypeStruct((_NC, _N, _HH), _F32),
        scratch_shapes=[pltpu.VMEM((3, _H), _F32)],
    )(
        raw2,
        deg_rep,
        g.reshape(1, _H),
        be.reshape(1, _H),
        w,
        b.reshape(1, _H),
    )


def _tc_pool_pred_body(
    raw_ref, deg_ref, g_ref, be_ref, batch_ref, w1_ref, b1_ref,
    w2_ref, b2_ref, out_ref, stats_ref, pool_ref,
):
    p = pl.program_id(0)
    i = pl.program_id(1)
    rsq = 1.0 / jnp.sqrt(deg_ref[0] + deg_ref[1])
    y0 = raw_ref[0] * rsq
    y1 = raw_ref[1] * rsq

    @pl.when(p == 0)
    def _():
        _stats_update(stats_ref, i, y0, y1)

    @pl.when(p == 1)
    def _():
        scale, shift = _bn_coeffs(stats_ref[...], g_ref[...], be_ref[...])
        h0 = jnp.maximum(y0 * scale[:, :_HH] + shift[:, :_HH], 0.0)
        h1 = jnp.maximum(y1 * scale[:, _HH:] + shift[:, _HH:], 0.0)
        seg = batch_ref[0]
        oh = (lax.broadcasted_iota(jnp.int32, (_NG, _BN), 0) == seg).astype(_F32)
        upd = jnp.concatenate([_dot(oh, h0), _dot(oh, h1)], axis=1)

        @pl.when(i == 0)
        def _():
            pool_ref[...] = upd

        @pl.when(i > 0)
        def _():
            pool_ref[...] = pool_ref[...] + upd

        @pl.when(i == _NB - 1)
        def _():
            hp = jnp.maximum(_dot(pool_ref[...], w1_ref[...]) + b1_ref[...], 0.0)
            out_ref[...] = _dot(hp, w2_ref[...]) + b2_ref[...]


def _tc_pool_pred(raw2, deg_rep, g, be, batch3, w1, b1, w2p, b2p):
    return pl.pallas_call(
        _tc_pool_pred_body,
        grid=(2, _NB),
        in_specs=[
            pl.BlockSpec((_NC, _BN, _HH), lambda p, i: (0, i, 0)),
            pl.BlockSpec((_NC, _BN, _HH), lambda p, i: (0, i, 0)),
            pl.BlockSpec((1, _H), lambda p, i: (0, 0)),
            pl.BlockSpec((1, _H), lambda p, i: (0, 0)),
            pl.BlockSpec((1, 1, _BN), lambda p, i: (i, 0, 0)),
            pl.BlockSpec((_H, _H), lambda p, i: (0, 0)),
            pl.BlockSpec((1, _H), lambda p, i: (0, 0)),
            pl.BlockSpec((_H, _HH), lambda p, i: (0, 0)),
            pl.BlockSpec((1, _HH), lambda p, i: (0, 0)),
        ],
        out_specs=pl.BlockSpec((_NG, _HH), lambda p, i: (0, 0)),
        out_shape=jax.ShapeDtypeStruct((_NG, _HH), _F32),
        scratch_shapes=[pltpu.VMEM((3, _H), _F32), pltpu.VMEM((_NG, _H), _F32)],
    )(
        raw2,
        deg_rep,
        g.reshape(1, _H),
        be.reshape(1, _H),
        batch3,
        w1,
        b1.reshape(1, _H),
        w2p,
        b2p.reshape(1, _HH),
    )


def kernel(x, edge_index, batch, atom_W, atom_b,
           conv_W0, conv_b0, bn_g0, bn_b0,
           conv_W1, conv_b1, bn_g1, bn_b1,
           conv_W2, conv_b2, bn_g2, bn_b2,
           pred_W1, pred_b1, pred_W2, pred_b2):
    src = edge_index[0].astype(jnp.int32).reshape(_NT, _EPT)
    dst = edge_index[1].astype(jnp.int32).reshape(_NT, _EPT)
    pad_src = (jnp.arange(_NPAD, dtype=jnp.int32)[None, :] * 61
               + jnp.arange(_NT, dtype=jnp.int32)[:, None] * 607) % _N
    pad_dst = _N + (jnp.arange(_NPAD, dtype=jnp.int32)[None, :] % _NTRASH
                    ) + jnp.zeros((_NT, 1), jnp.int32)
    srcp = jnp.concatenate([src, pad_src], axis=1)
    dstp = jnp.concatenate([dst, pad_dst], axis=1)
    packed3 = (dstp * _PACK + srcp).reshape(_NT, _NCHP, _CHP)
    dstp3 = dstp.reshape(_NT, _NCHP, _CHP)
    ones_ch = jnp.ones((_CHP, _HH), _F32)
    init2 = jnp.concatenate(
        [jnp.ones((_RPT, _HH), _F32), jnp.zeros((_RPT, _HH), _F32)], axis=0
    )
    batch3 = batch.astype(jnp.int32).reshape(_NB, 1, _BN)
    w2p = jnp.pad(pred_W2, ((0, 0), (0, _HH - pred_W2.shape[1])))
    b2p = jnp.pad(pred_b2, (0, _HH - pred_b2.shape[0]))

    deg_rep = _sc_degree(dstp3, ones_ch, init2).reshape(_NC, _N, _HH)

    hs = _tc_l0(x, atom_W, atom_b, conv_W0, conv_b0, deg_rep)
    raw = _sc_edge(hs.reshape(_NC * _N, _HH), packed3).reshape(_NC, _N, _HH)
    hs = _tc_bn_layer(raw, deg_rep, bn_g0, bn_b0, conv_W1, conv_b1)
    raw = _sc_edge(hs.reshape(_NC * _N, _HH), packed3).reshape(_NC, _N, _HH)
    hs = _tc_bn_layer(raw, deg_rep, bn_g1, bn_b1, conv_W2, conv_b2)
    raw = _sc_edge(hs.reshape(_NC * _N, _HH), packed3).reshape(_NC, _N, _HH)
    y = _tc_pool_pred(raw, deg_rep, bn_g2, bn_b2, batch3,
                      pred_W1, pred_b1, w2p, b2p)
    return y[:, 0]
